# baseline (speedup 1.0000x reference)
import functools

import jax
import jax.numpy as jnp
from jax import lax
from jax.experimental import pallas as pl
from jax.experimental.pallas import tpu as pltpu
from jax.experimental.pallas import tpu_sc as plsc

B = 16
T = 1024
K = 4096
D = 64
HOP = 128
C = B * T
F = 4 * C

NC = 2
NS = 16
NW = NC * NS
C_W = C // NW
F_W = 4 * C_W
NSUB = 4
C_SUB = C_W // NSUB
F_SUB = 4 * C_SUB
SEG = 128
NSEG = F_SUB // SEG
NSEG_W = F_W // SEG


def _sc_gather_combine(ids_flat, cb_all):
    mesh = plsc.VectorSubcoreMesh(core_axis_name="c", subcore_axis_name="s")

    @functools.partial(
        pl.kernel,
        out_type=jax.ShapeDtypeStruct((F, 2 * D), jnp.float32),
        mesh=mesh,
        scratch_types=[
            pltpu.VMEM((C_W * 7,), jnp.int32),
            pltpu.VMEM((3, NSEG_W, SEG), jnp.int32),
            pltpu.VMEM((2, F_SUB, D), jnp.float32),
            pltpu.SemaphoreType.DMA,
            pltpu.SemaphoreType.DMA,
            pltpu.SemaphoreType.DMA,
            pltpu.SemaphoreType.DMA,
            pltpu.SemaphoreType.DMA,
        ],
        compiler_params=pltpu.CompilerParams(needs_layout_passes=False,
                                             use_tc_tiling_on_sc=False),
    )
    def k(ids_hbm, cb_hbm, z_hbm, ids_v, idx_v, z_v,
          sem_ids, sem_init0, sem_init1, sem_add, sem_out):
        wid = lax.axis_index("s") * NC + lax.axis_index("c")
        pltpu.async_copy(ids_hbm.at[pl.ds(wid * C_W * 7, C_W * 7)],
                         ids_v, sem_ids).wait()
        lane = lax.broadcasted_iota(jnp.int32, (16,), 0)
        for i in range(F_W // 16):
            f = lane + i * 16
            t = f >> 2
            s = f & 3
            base7 = t * 7
            seg, off = divmod(i * 16, SEG)
            idx_v[0, seg, pl.ds(off, 16)] = plsc.load_gather(ids_v, [base7])
            idx_v[1, seg, pl.ds(off, 16)] = plsc.load_gather(
                ids_v, [base7 + (1 + (s >> 1))])
            idx_v[2, seg, pl.ds(off, 16)] = plsc.load_gather(
                ids_v, [base7 + (3 + s)])

        def fire(level, sub, buf, add, sem):
            return [pltpu.async_copy(
                        cb_hbm.at[idx_v.at[level, NSEG * sub + g]],
                        z_v.at[buf, pl.ds(g * SEG, SEG)],
                        sem, add=add)
                    for g in range(NSEG)]

        def fire_out(sub, buf):
            base_f = wid * F_W + sub * F_SUB
            return pltpu.async_copy(
                z_v.at[buf],
                z_hbm.at[pl.ds(base_f, F_SUB), pl.ds(0, D)], sem_out)

        sem_init = (sem_init0, sem_init1)
        init_d = [None] * NSUB
        out_d = [None] * NSUB
        init_d[0] = fire(2, 0, 0, False, sem_init[0])
        for sub in range(NSUB):
            buf = sub % 2
            for dsc in init_d[sub]:
                dsc.wait()
            if sub + 1 < NSUB:
                if sub >= 1:
                    out_d[sub - 1].wait()
                init_d[sub + 1] = fire(2, sub + 1, 1 - buf, False,
                                       sem_init[1 - buf])
            for dsc in fire(1, sub, buf, True, sem_add):
                dsc.wait()
            for dsc in fire(0, sub, buf, True, sem_add):
                dsc.wait()
            out_d[sub] = fire_out(sub, buf)
        out_d[NSUB - 2].wait()
        out_d[NSUB - 1].wait()

    return k(ids_flat, cb_all)


def _tc_decode(z, W_dec, b_dec):
    ROWS = 4096

    def body(z_ref, w_ref, b_ref, o_ref):
        acc = jnp.dot(z_ref[:, :D], w_ref[...],
                      preferred_element_type=jnp.float32)
        o_ref[...] = jnp.tanh(acc + b_ref[...])

    return pl.pallas_call(
        body,
        grid=(F // ROWS,),
        in_specs=[
            pl.BlockSpec((ROWS, 2 * D), lambda i: (i, 0)),
            pl.BlockSpec((D, HOP), lambda i: (0, 0)),
            pl.BlockSpec((1, HOP), lambda i: (0, 0)),
        ],
        out_specs=pl.BlockSpec((ROWS, HOP), lambda i: (i, 0)),
        out_shape=jax.ShapeDtypeStruct((F, HOP), jnp.float32),
    )(z, W_dec, b_dec.reshape(1, HOP))


def kernel(ids, cb1, cb2, cb3, W_dec, b_dec):
    ids_flat = ids.reshape(-1).astype(jnp.int32)
    cb_all = jnp.concatenate([cb1, cb2, cb3], axis=0)
    z = _sc_gather_combine(ids_flat, cb_all)
    out = _tc_decode(z, W_dec, b_dec)
    return out.reshape(B, 1, 4 * T * HOP)


# speedup vs baseline: 1.1319x; 1.1319x over previous
import functools

import jax
import jax.numpy as jnp
from jax import lax
from jax.experimental import pallas as pl
from jax.experimental.pallas import tpu as pltpu
from jax.experimental.pallas import tpu_sc as plsc

B = 16
T = 1024
K = 4096
D = 64
HOP = 128
C = B * T
F = 4 * C

NC = 2
NS = 16
NW = NC * NS
C_W = C // NW
F_W = 4 * C_W
NSUB = 4
C_SUB = C_W // NSUB
F_SUB = 4 * C_SUB
SEG = 128
NSEG = F_SUB // SEG
NSEG_W = F_W // SEG


def _sc_gather_combine(ids_flat, cb1, cb2, cb3):
    mesh = plsc.VectorSubcoreMesh(core_axis_name="c", subcore_axis_name="s")

    @functools.partial(
        pl.kernel,
        out_type=jax.ShapeDtypeStruct((F, 2 * D), jnp.float32),
        mesh=mesh,
        scratch_types=[
            pltpu.VMEM((C_W * 7,), jnp.int32),
            pltpu.VMEM((3, NSEG_W, SEG), jnp.int32),
            pltpu.VMEM((2, F_SUB, D), jnp.float32),
            pltpu.SemaphoreType.DMA,
            pltpu.SemaphoreType.DMA,
            pltpu.SemaphoreType.DMA,
            pltpu.SemaphoreType.DMA,
            pltpu.SemaphoreType.DMA,
        ],
        compiler_params=pltpu.CompilerParams(needs_layout_passes=False,
                                             use_tc_tiling_on_sc=False),
    )
    def k(ids_hbm, cb1_hbm, cb2_hbm, cb3_hbm, z_hbm, ids_v, idx_v, z_v,
          sem_ids, sem_init0, sem_init1, sem_add, sem_out):
        wid = lax.axis_index("s") * NC + lax.axis_index("c")
        pltpu.async_copy(ids_hbm.at[pl.ds(wid * C_W * 7, C_W * 7)],
                         ids_v, sem_ids).wait()
        lane = lax.broadcasted_iota(jnp.int32, (16,), 0)
        for i in range(F_W // 16):
            f = lane + i * 16
            t = f >> 2
            s = f & 3
            base7 = t * 7
            seg, off = divmod(i * 16, SEG)
            idx_v[0, seg, pl.ds(off, 16)] = plsc.load_gather(ids_v, [base7])
            idx_v[1, seg, pl.ds(off, 16)] = plsc.load_gather(
                ids_v, [base7 + (1 + (s >> 1))]) - K
            idx_v[2, seg, pl.ds(off, 16)] = plsc.load_gather(
                ids_v, [base7 + (3 + s)]) - 2 * K

        def fire(level, sub, buf, add, sem):
            cb_hbm = (cb1_hbm, cb2_hbm, cb3_hbm)[level]
            return [pltpu.async_copy(
                        cb_hbm.at[idx_v.at[level, NSEG * sub + g]],
                        z_v.at[buf, pl.ds(g * SEG, SEG)],
                        sem, add=add)
                    for g in range(NSEG)]

        def fire_out(sub, buf):
            base_f = wid * F_W + sub * F_SUB
            return pltpu.async_copy(
                z_v.at[buf],
                z_hbm.at[pl.ds(base_f, F_SUB), pl.ds(0, D)], sem_out)

        sem_init = (sem_init0, sem_init1)
        init_d = [None] * NSUB
        out_d = [None] * NSUB
        init_d[0] = fire(2, 0, 0, False, sem_init[0])
        for sub in range(NSUB):
            buf = sub % 2
            for dsc in init_d[sub]:
                dsc.wait()
            if sub + 1 < NSUB:
                if sub >= 1:
                    out_d[sub - 1].wait()
                init_d[sub + 1] = fire(2, sub + 1, 1 - buf, False,
                                       sem_init[1 - buf])
            for dsc in fire(1, sub, buf, True, sem_add):
                dsc.wait()
            for dsc in fire(0, sub, buf, True, sem_add):
                dsc.wait()
            out_d[sub] = fire_out(sub, buf)
        out_d[NSUB - 2].wait()
        out_d[NSUB - 1].wait()

    return k(ids_flat, cb1, cb2, cb3)


def _tc_decode(z, W_dec, b_dec):
    ROWS = 8192

    def body(z_ref, w_ref, b_ref, o_ref):
        acc = jnp.dot(z_ref[:, :D], w_ref[...],
                      preferred_element_type=jnp.float32)
        o_ref[...] = jnp.tanh(acc + b_ref[...])

    return pl.pallas_call(
        body,
        grid=(F // ROWS,),
        in_specs=[
            pl.BlockSpec((ROWS, 2 * D), lambda i: (i, 0)),
            pl.BlockSpec((D, HOP), lambda i: (0, 0)),
            pl.BlockSpec((1, HOP), lambda i: (0, 0)),
        ],
        out_specs=pl.BlockSpec((ROWS, HOP), lambda i: (i, 0)),
        out_shape=jax.ShapeDtypeStruct((F, HOP), jnp.float32),
    )(z, W_dec, b_dec.reshape(1, HOP))


def kernel(ids, cb1, cb2, cb3, W_dec, b_dec):
    ids_flat = ids.reshape(-1).astype(jnp.int32)
    z = _sc_gather_combine(ids_flat, cb1, cb2, cb3)
    out = _tc_decode(z, W_dec, b_dec)
    return out.reshape(B, 1, 4 * T * HOP)


# speedup vs baseline: 1.2842x; 1.1346x over previous
import functools

import jax
import jax.numpy as jnp
from jax import lax
from jax.experimental import pallas as pl
from jax.experimental.pallas import tpu as pltpu
from jax.experimental.pallas import tpu_sc as plsc

B = 16
T = 1024
K = 4096
D = 64
HOP = 128
C = B * T
F = 4 * C

NC = 2
NS = 16
NW = NC * NS
C_W = C // NW
F_W = 4 * C_W
NSUB = 8
C_SUB = C_W // NSUB
F_SUB = 4 * C_SUB
SEG = 128
NSEG = F_SUB // SEG
NSEG_W = F_W // SEG


def _sc_gather_combine(ids_flat, cb1, cb2, cb3):
    mesh = plsc.VectorSubcoreMesh(core_axis_name="c", subcore_axis_name="s")

    H_SUB = F_SUB // 2
    NSEG2 = F_W // 2 // SEG
    NSEG2_SUB = NSEG2 // NSUB or 1

    @functools.partial(
        pl.kernel,
        out_type=jax.ShapeDtypeStruct((F, 2 * D), jnp.float32),
        mesh=mesh,
        scratch_types=[
            pltpu.VMEM((C_W * 7,), jnp.int32),
            pltpu.VMEM((NSEG_W, SEG), jnp.int32),
            pltpu.VMEM((NSEG2, SEG), jnp.int32),
            pltpu.VMEM((NSUB, C_SUB), jnp.int32),
            pltpu.VMEM((2, F_SUB, D), jnp.float32),
            pltpu.VMEM((2, H_SUB, D), jnp.float32),
            pltpu.VMEM((2, C_SUB, D), jnp.float32),
            pltpu.SemaphoreType.DMA,
            pltpu.SemaphoreType.DMA,
            pltpu.SemaphoreType.DMA,
            pltpu.SemaphoreType.DMA,
            pltpu.SemaphoreType.DMA,
        ],
        compiler_params=pltpu.CompilerParams(needs_layout_passes=False,
                                             use_tc_tiling_on_sc=False),
    )
    def k(ids_hbm, cb1_hbm, cb2_hbm, cb3_hbm, z_hbm,
          ids_v, idx3_v, idx2_v, idx1_v, z_v, l2_v, l1_v,
          sem_ids, sem_g0, sem_g1, sem_aux, sem_out):
        wid = lax.axis_index("s") * NC + lax.axis_index("c")
        pltpu.async_copy(ids_hbm.at[pl.ds(wid * C_W * 7, C_W * 7)],
                         ids_v, sem_ids).wait()
        lane = lax.broadcasted_iota(jnp.int32, (16,), 0)
        for i in range(F_W // 16):
            f = lane + i * 16
            s = f & 3
            seg, off = divmod(i * 16, SEG)
            idx3_v[seg, pl.ds(off, 16)] = plsc.load_gather(
                ids_v, [(f >> 2) * 7 + (3 + s)]) - 2 * K
        for i in range(F_W // 2 // 16):
            h = lane + i * 16
            seg, off = divmod(i * 16, SEG)
            idx2_v[seg, pl.ds(off, 16)] = plsc.load_gather(
                ids_v, [(h >> 1) * 7 + (1 + (h & 1))]) - K
        for i in range(F_W // 4 // 16):
            t = lane + i * 16
            seg, off = divmod(i * 16, C_SUB)
            idx1_v[seg, pl.ds(off, 16)] = plsc.load_gather(ids_v, [t * 7])

        sem_g = (sem_g0, sem_g1)

        def fire(sub, buf):
            ds_ = [pltpu.async_copy(
                       cb3_hbm.at[idx3_v.at[NSEG * sub + g]],
                       z_v.at[buf, pl.ds(g * SEG, SEG)],
                       sem_g[buf])
                   for g in range(NSEG)]
            ds_ += [pltpu.async_copy(
                        cb2_hbm.at[idx2_v.at[NSEG2_SUB * sub + g]],
                        l2_v.at[buf, pl.ds(g * SEG, SEG)],
                        sem_aux)
                    for g in range(NSEG2_SUB)]
            ds_.append(pltpu.async_copy(
                cb1_hbm.at[idx1_v.at[sub]], l1_v.at[buf], sem_aux))
            return ds_

        def fire_out(sub, buf):
            base_f = wid * F_W + sub * F_SUB
            return pltpu.async_copy(
                z_v.at[buf],
                z_hbm.at[pl.ds(base_f, F_SUB), pl.ds(0, D)], sem_out)

        def add_phase(buf):
            def body(tc, carry):
                a = [l1_v[buf, tc, pl.ds(c * 16, 16)] for c in range(4)]
                for u in range(2):
                    acc = [a[c] + l2_v[buf, 2 * tc + u, pl.ds(c * 16, 16)]
                           for c in range(4)]
                    for s2 in range(2):
                        fr = 4 * tc + 2 * u + s2
                        for c in range(4):
                            plsc.addupdate(
                                z_v.at[buf, fr, pl.ds(c * 16, 16)], acc[c])
                return carry
            lax.fori_loop(0, C_SUB, body, 0)

        gat_d = [None] * NSUB
        out_d = [None] * NSUB
        gat_d[0] = fire(0, 0)
        for sub in range(NSUB):
            buf = sub % 2
            for dsc in gat_d[sub]:
                dsc.wait()
            if sub + 1 < NSUB:
                if sub >= 1:
                    out_d[sub - 1].wait()
                gat_d[sub + 1] = fire(sub + 1, 1 - buf)
            add_phase(buf)
            out_d[sub] = fire_out(sub, buf)
        out_d[NSUB - 2].wait()
        out_d[NSUB - 1].wait()

    return k(ids_flat, cb1, cb2, cb3)


def _tc_decode(z, W_dec, b_dec):
    ROWS = 8192

    def body(z_ref, w_ref, b_ref, o_ref):
        acc = jnp.dot(z_ref[:, :D], w_ref[...],
                      preferred_element_type=jnp.float32)
        o_ref[...] = jnp.tanh(acc + b_ref[...])

    return pl.pallas_call(
        body,
        grid=(F // ROWS,),
        in_specs=[
            pl.BlockSpec((ROWS, 2 * D), lambda i: (i, 0)),
            pl.BlockSpec((D, HOP), lambda i: (0, 0)),
            pl.BlockSpec((1, HOP), lambda i: (0, 0)),
        ],
        out_specs=pl.BlockSpec((ROWS, HOP), lambda i: (i, 0)),
        out_shape=jax.ShapeDtypeStruct((F, HOP), jnp.float32),
    )(z, W_dec, b_dec.reshape(1, HOP))


def kernel(ids, cb1, cb2, cb3, W_dec, b_dec):
    ids_flat = ids.reshape(-1).astype(jnp.int32)
    z = _sc_gather_combine(ids_flat, cb1, cb2, cb3)
    out = _tc_decode(z, W_dec, b_dec)
    return out.reshape(B, 1, 4 * T * HOP)


# speedup vs baseline: 1.4715x; 1.1458x over previous
import functools

import jax
import jax.numpy as jnp
from jax import lax
from jax.experimental import pallas as pl
from jax.experimental.pallas import tpu as pltpu
from jax.experimental.pallas import tpu_sc as plsc

B = 16
T = 1024
K = 4096
D = 64
HOP = 128
C = B * T
F = 4 * C

NC = 2
NS = 16
NW = NC * NS
C_W = C // NW
F_W = 4 * C_W
NSUB = 8
C_SUB = C_W // NSUB
F_SUB = 4 * C_SUB
SEG = 128
NSEG = F_SUB // SEG
NSEG_W = F_W // SEG


def _sc_gather_combine(ids_flat, cb1, cb2, cb3):
    mesh = plsc.VectorSubcoreMesh(core_axis_name="c", subcore_axis_name="s")

    H_SUB = F_SUB // 2
    NSEG2 = F_W // 2 // SEG
    NSEG2_SUB = NSEG2 // NSUB or 1

    @functools.partial(
        pl.kernel,
        out_type=jax.ShapeDtypeStruct((F, 2 * D), jnp.float32),
        mesh=mesh,
        scratch_types=[
            pltpu.VMEM((C_W * 7,), jnp.int32),
            pltpu.VMEM((NSEG_W, SEG), jnp.int32),
            pltpu.VMEM((NSEG2, SEG), jnp.int32),
            pltpu.VMEM((NSUB, C_SUB), jnp.int32),
            pltpu.VMEM((2, F_SUB, D), jnp.float32),
            pltpu.VMEM((2, H_SUB, D), jnp.float32),
            pltpu.VMEM((2, C_SUB, D), jnp.float32),
            pltpu.SemaphoreType.DMA,
            pltpu.SemaphoreType.DMA,
            pltpu.SemaphoreType.DMA,
            pltpu.SemaphoreType.DMA,
            pltpu.SemaphoreType.DMA,
        ],
        compiler_params=pltpu.CompilerParams(needs_layout_passes=False,
                                             use_tc_tiling_on_sc=False),
    )
    def k(ids_hbm, cb1_hbm, cb2_hbm, cb3_hbm, z_hbm,
          ids_v, idx3_v, idx2_v, idx1_v, z_v, l2_v, l1_v,
          sem_ids, sem_g0, sem_g1, sem_aux, sem_out):
        wid = lax.axis_index("s") * NC + lax.axis_index("c")
        pltpu.async_copy(ids_hbm.at[pl.ds(wid * C_W * 7, C_W * 7)],
                         ids_v, sem_ids).wait()
        lane = lax.broadcasted_iota(jnp.int32, (16,), 0)

        def build3(i, carry):
            f = lane + i * 16
            s = f & 3
            idx3_v[i >> 3, pl.ds((i & 7) * 16, 16)] = plsc.load_gather(
                ids_v, [(f >> 2) * 7 + (3 + s)]) - 2 * K
            return carry

        def build2(i, carry):
            h = lane + i * 16
            idx2_v[i >> 3, pl.ds((i & 7) * 16, 16)] = plsc.load_gather(
                ids_v, [(h >> 1) * 7 + (1 + (h & 1))]) - K
            return carry

        def build1(i, carry):
            t = lane + i * 16
            idx1_v[i >> 2, pl.ds((i & 3) * 16, 16)] = plsc.load_gather(
                ids_v, [t * 7])
            return carry

        lax.fori_loop(0, F_W // 16, build3, 0)
        lax.fori_loop(0, F_W // 2 // 16, build2, 0)
        lax.fori_loop(0, F_W // 4 // 16, build1, 0)

        sem_g = (sem_g0, sem_g1)

        def fire(sub, buf):
            ds_ = [pltpu.async_copy(
                       cb3_hbm.at[idx3_v.at[NSEG * sub + g]],
                       z_v.at[buf, pl.ds(g * SEG, SEG)],
                       sem_g[buf])
                   for g in range(NSEG)]
            ds_ += [pltpu.async_copy(
                        cb2_hbm.at[idx2_v.at[NSEG2_SUB * sub + g]],
                        l2_v.at[buf, pl.ds(g * SEG, SEG)],
                        sem_aux)
                    for g in range(NSEG2_SUB)]
            ds_.append(pltpu.async_copy(
                cb1_hbm.at[idx1_v.at[sub]], l1_v.at[buf], sem_aux))
            return ds_

        def fire_out(sub, buf):
            base_f = wid * F_W + sub * F_SUB
            return pltpu.async_copy(
                z_v.at[buf],
                z_hbm.at[pl.ds(base_f, F_SUB), pl.ds(0, D)], sem_out)

        def add_phase(buf):
            def body(tc, carry):
                a = [l1_v[buf, tc, pl.ds(c * 16, 16)] for c in range(4)]
                for u in range(2):
                    acc = [a[c] + l2_v[buf, 2 * tc + u, pl.ds(c * 16, 16)]
                           for c in range(4)]
                    for s2 in range(2):
                        fr = 4 * tc + 2 * u + s2
                        for c in range(4):
                            plsc.addupdate(
                                z_v.at[buf, fr, pl.ds(c * 16, 16)], acc[c])
                return carry
            lax.fori_loop(0, C_SUB, body, 0)

        gat_d = [None] * NSUB
        out_d = [None] * NSUB
        gat_d[0] = fire(0, 0)
        for sub in range(NSUB):
            buf = sub % 2
            for dsc in gat_d[sub]:
                dsc.wait()
            if sub + 1 < NSUB:
                if sub >= 1:
                    out_d[sub - 1].wait()
                gat_d[sub + 1] = fire(sub + 1, 1 - buf)
            add_phase(buf)
            out_d[sub] = fire_out(sub, buf)
        out_d[NSUB - 2].wait()
        out_d[NSUB - 1].wait()

    return k(ids_flat, cb1, cb2, cb3)


def _tc_decode(z, W_dec, b_dec):
    ROWS = 8192

    def body(z_ref, w_ref, b_ref, o_ref):
        acc = jnp.dot(z_ref[:, :D], w_ref[...],
                      preferred_element_type=jnp.float32)
        o_ref[...] = jnp.tanh(acc + b_ref[...])

    return pl.pallas_call(
        body,
        grid=(F // ROWS,),
        in_specs=[
            pl.BlockSpec((ROWS, 2 * D), lambda i: (i, 0)),
            pl.BlockSpec((D, HOP), lambda i: (0, 0)),
            pl.BlockSpec((1, HOP), lambda i: (0, 0)),
        ],
        out_specs=pl.BlockSpec((ROWS, HOP), lambda i: (i, 0)),
        out_shape=jax.ShapeDtypeStruct((F, HOP), jnp.float32),
    )(z, W_dec, b_dec.reshape(1, HOP))


def kernel(ids, cb1, cb2, cb3, W_dec, b_dec):
    ids_flat = ids.reshape(-1).astype(jnp.int32)
    z = _sc_gather_combine(ids_flat, cb1, cb2, cb3)
    out = _tc_decode(z, W_dec, b_dec)
    return out.reshape(B, 1, 4 * T * HOP)


# speedup vs baseline: 1.4959x; 1.0166x over previous
import functools

import jax
import jax.numpy as jnp
from jax import lax
from jax.experimental import pallas as pl
from jax.experimental.pallas import tpu as pltpu
from jax.experimental.pallas import tpu_sc as plsc

B = 16
T = 1024
K = 4096
D = 64
HOP = 128
C = B * T
F = 4 * C

NC = 2
NS = 16
NW = NC * NS
C_W = C // NW
F_W = 4 * C_W
NSUB = 8
C_SUB = C_W // NSUB
F_SUB = 4 * C_SUB
SEG = 128
NSEG = F_SUB // SEG
NSEG_W = F_W // SEG


def _sc_gather_combine(ids_flat, cb1, cb2, cb3):
    mesh = plsc.VectorSubcoreMesh(core_axis_name="c", subcore_axis_name="s")

    H_SUB = F_SUB // 2
    NSEG2 = F_W // 2 // SEG
    NSEG2_SUB = NSEG2 // NSUB or 1

    @functools.partial(
        pl.kernel,
        out_type=jax.ShapeDtypeStruct((F, 2 * D), jnp.float32),
        mesh=mesh,
        scratch_types=[
            pltpu.VMEM((C_W * 7,), jnp.int32),
            pltpu.VMEM((NSEG_W, SEG), jnp.int32),
            pltpu.VMEM((NSEG2, SEG), jnp.int32),
            pltpu.VMEM((NSUB, C_SUB), jnp.int32),
            pltpu.VMEM((2, F_SUB, D), jnp.float32),
            pltpu.VMEM((2, H_SUB, D), jnp.float32),
            pltpu.VMEM((2, C_SUB, D), jnp.float32),
            pltpu.SemaphoreType.DMA,
            pltpu.SemaphoreType.DMA,
            pltpu.SemaphoreType.DMA,
            pltpu.SemaphoreType.DMA,
            pltpu.SemaphoreType.DMA,
        ],
        compiler_params=pltpu.CompilerParams(needs_layout_passes=False,
                                             use_tc_tiling_on_sc=False),
    )
    def k(ids_hbm, cb1_hbm, cb2_hbm, cb3_hbm, z_hbm,
          ids_v, idx3_v, idx2_v, idx1_v, z_v, l2_v, l1_v,
          sem_ids, sem_g0, sem_g1, sem_aux, sem_out):
        wid = lax.axis_index("s") * NC + lax.axis_index("c")
        pltpu.async_copy(ids_hbm.at[pl.ds(wid * C_W * 7, C_W * 7)],
                         ids_v, sem_ids).wait()
        lane = lax.broadcasted_iota(jnp.int32, (16,), 0)

        def build3(i, carry):
            f = lane + i * 16
            s = f & 3
            idx3_v[i >> 3, pl.ds((i & 7) * 16, 16)] = plsc.load_gather(
                ids_v, [(f >> 2) * 7 + (3 + s)]) - 2 * K
            return carry

        def build2(i, carry):
            h = lane + i * 16
            idx2_v[i >> 3, pl.ds((i & 7) * 16, 16)] = plsc.load_gather(
                ids_v, [(h >> 1) * 7 + (1 + (h & 1))]) - K
            return carry

        def build1(i, carry):
            t = lane + i * 16
            idx1_v[i >> 2, pl.ds((i & 3) * 16, 16)] = plsc.load_gather(
                ids_v, [t * 7])
            return carry

        lax.fori_loop(0, F_W // 16, build3, 0)
        lax.fori_loop(0, F_W // 2 // 16, build2, 0)
        lax.fori_loop(0, F_W // 4 // 16, build1, 0)

        sem_g = (sem_g0, sem_g1)

        def fire(sub, buf):
            ds_ = [pltpu.async_copy(
                       cb3_hbm.at[idx3_v.at[NSEG * sub + g]],
                       z_v.at[buf, pl.ds(g * SEG, SEG)],
                       sem_g[buf])
                   for g in range(NSEG)]
            ds_ += [pltpu.async_copy(
                        cb2_hbm.at[idx2_v.at[NSEG2_SUB * sub + g]],
                        l2_v.at[buf, pl.ds(g * SEG, SEG)],
                        sem_aux)
                    for g in range(NSEG2_SUB)]
            ds_.append(pltpu.async_copy(
                cb1_hbm.at[idx1_v.at[sub]], l1_v.at[buf], sem_aux))
            return ds_

        def fire_out(sub, buf):
            base_f = wid * F_W + sub * F_SUB
            return pltpu.async_copy(
                z_v.at[buf],
                z_hbm.at[pl.ds(base_f, F_SUB), pl.ds(0, D)], sem_out)

        def add_phase(buf):
            def body(tc, carry):
                a = [l1_v[buf, tc, pl.ds(c * 16, 16)] for c in range(4)]
                for u in range(2):
                    acc = [a[c] + l2_v[buf, 2 * tc + u, pl.ds(c * 16, 16)]
                           for c in range(4)]
                    for s2 in range(2):
                        fr = 4 * tc + 2 * u + s2
                        for c in range(4):
                            plsc.addupdate(
                                z_v.at[buf, fr, pl.ds(c * 16, 16)], acc[c])
                return carry
            lax.fori_loop(0, C_SUB, body, 0)

        gat_d = [None] * NSUB
        out_d = [None] * NSUB
        gat_d[0] = fire(0, 0)
        for sub in range(NSUB):
            buf = sub % 2
            for dsc in gat_d[sub]:
                dsc.wait()
            if sub + 1 < NSUB:
                if sub >= 1:
                    out_d[sub - 1].wait()
                gat_d[sub + 1] = fire(sub + 1, 1 - buf)
            add_phase(buf)
            out_d[sub] = fire_out(sub, buf)
        out_d[NSUB - 2].wait()
        out_d[NSUB - 1].wait()

    return k(ids_flat, cb1, cb2, cb3)


def _tc_decode(z, W_dec, b_dec):
    ROWS = 16384

    def body(z_ref, w_ref, b_ref, o_ref):
        acc = jnp.dot(z_ref[:, :D], w_ref[...],
                      preferred_element_type=jnp.float32)
        o_ref[...] = jnp.tanh(acc + b_ref[...])

    return pl.pallas_call(
        body,
        grid=(F // ROWS,),
        in_specs=[
            pl.BlockSpec((ROWS, 2 * D), lambda i: (i, 0)),
            pl.BlockSpec((D, HOP), lambda i: (0, 0)),
            pl.BlockSpec((1, HOP), lambda i: (0, 0)),
        ],
        out_specs=pl.BlockSpec((ROWS, HOP), lambda i: (i, 0)),
        out_shape=jax.ShapeDtypeStruct((F, HOP), jnp.float32),
    )(z, W_dec, b_dec.reshape(1, HOP))


def kernel(ids, cb1, cb2, cb3, W_dec, b_dec):
    ids_flat = ids.reshape(-1).astype(jnp.int32)
    z = _sc_gather_combine(ids_flat, cb1, cb2, cb3)
    out = _tc_decode(z, W_dec, b_dec)
    return out.reshape(B, 1, 4 * T * HOP)
